# baseline (speedup 1.0000x reference)
import functools

import jax
import jax.numpy as jnp
from jax import lax
from jax.experimental import pallas as pl
from jax.experimental.pallas import tpu as pltpu
from jax.experimental.pallas import tpu_sc as plsc

NUM_CORES = 2
NUM_SUBCORES = 16
LANES = 16
NUM_WORKERS = NUM_CORES * NUM_SUBCORES
CHUNK = 16
NBUF = 4


def kernel(x, pe_weight):
    batch, seq_len, d = x.shape
    seq_per_w = seq_len // NUM_WORKERS
    steps = seq_per_w // CHUNK
    total = steps * batch
    words = CHUNK * d
    assert seq_len % NUM_WORKERS == 0 and seq_per_w % CHUNK == 0
    assert total > 2 * NBUF

    xf = x.reshape(batch * seq_len * d)
    pef = pe_weight.reshape(pe_weight.shape[0] * d)

    mesh = plsc.VectorSubcoreMesh(
        core_axis_name="c", subcore_axis_name="s",
        num_cores=NUM_CORES, num_subcores=NUM_SUBCORES,
    )

    @functools.partial(
        pl.kernel,
        out_type=jax.ShapeDtypeStruct((batch * seq_len * d,), x.dtype),
        mesh=mesh,
        scratch_types=[
            pltpu.VMEM((NBUF, words), jnp.float32),
            pltpu.VMEM((2, words), jnp.float32),
            pltpu.SemaphoreType.DMA((NBUF,)),
            pltpu.SemaphoreType.DMA((NBUF,)),
            pltpu.SemaphoreType.DMA((2,)),
        ],
    )
    def sc_add(x_hbm, pe_hbm, out_hbm, xb, peb, xsem, osem, psem):
        wid = lax.axis_index("s") * NUM_CORES + lax.axis_index("c")
        s_base = wid * seq_per_w

        def x_off(t):
            sblk, b = divmod(t, batch)
            return (b * seq_len + s_base + sblk * CHUNK) * d

        def xcopy(t):
            slot = t % NBUF
            return pltpu.make_async_copy(
                x_hbm.at[pl.ds(x_off(t), words)], xb.at[slot], xsem.at[slot])

        def ocopy(t):
            slot = t % NBUF
            return pltpu.make_async_copy(
                xb.at[slot], out_hbm.at[pl.ds(x_off(t), words)], osem.at[slot])

        def pecopy(sblk):
            slot = sblk % 2
            return pltpu.make_async_copy(
                pe_hbm.at[pl.ds((s_base + sblk * CHUNK) * d, words)],
                peb.at[slot], psem.at[slot])

        pecopy(0).start()
        xcopy(0).start()
        xcopy(1).start()

        for t in range(total):
            sblk, b = divmod(t, batch)
            slot = t % NBUF
            if b == 0:
                pecopy(sblk).wait()
                if sblk + 1 < steps:
                    pecopy(sblk + 1).start()
            xcopy(t).wait()

            pslot = sblk % 2

            @plsc.parallel_loop(0, words, LANES, unroll=8)
            def add_body(o):
                plsc.addupdate(
                    xb.at[slot, pl.ds(o, LANES)], peb[pslot, pl.ds(o, LANES)])

            ocopy(t).start()
            if t + 2 < total:
                if t - 2 >= 0:
                    ocopy(t - 2).wait()
                xcopy(t + 2).start()

        for t in range(total - NBUF, total):
            ocopy(t).wait()

    out = sc_add(xf, pef)
    return out.reshape(batch, seq_len, d)


# speedup vs baseline: 4.9262x; 4.9262x over previous
import jax
import jax.numpy as jnp
from jax.experimental import pallas as pl
from jax.experimental.pallas import tpu as pltpu

CHUNK = 1024
NBUF = 6
AHEAD = 4


def kernel(x, pe_weight):
    batch, seq_len, d = x.shape
    rows = batch * seq_len
    num_sblk = seq_len // CHUNK
    total = num_sblk * batch
    assert seq_len % CHUNK == 0 and total > NBUF

    x2 = x.reshape(rows, d)
    pe2 = pe_weight[:seq_len]

    def body(x_hbm, pe_hbm, o_hbm, xb, peb, xs, os_, ps):
        def row0(t):
            sblk, b = divmod(t, batch)
            return b * seq_len + sblk * CHUNK

        def x_copy(t):
            slot = t % NBUF
            return pltpu.make_async_copy(
                x_hbm.at[pl.ds(row0(t), CHUNK)], xb.at[slot], xs.at[slot])

        def pe_copy(sblk):
            slot = sblk % 2
            return pltpu.make_async_copy(
                pe_hbm.at[pl.ds(sblk * CHUNK, CHUNK)], peb.at[slot], ps.at[slot])

        def o_copy(t):
            slot = t % NBUF
            return pltpu.make_async_copy(
                xb.at[slot], o_hbm.at[pl.ds(row0(t), CHUNK)], os_.at[slot])

        pe_copy(0).start()
        if num_sblk > 1:
            pe_copy(1).start()
        for t in range(min(AHEAD, total)):
            x_copy(t).start()

        for t in range(total):
            sblk, b = divmod(t, batch)
            slot = t % NBUF
            if b == 0:
                pe_copy(sblk).wait()
            x_copy(t).wait()
            xb[slot] = xb[slot] + peb[sblk % 2]
            o_copy(t).start()
            if b == batch - 1 and sblk + 2 < num_sblk:
                pe_copy(sblk + 2).start()
            if t + AHEAD < total:
                if t + AHEAD - NBUF >= 0:
                    o_copy(t + AHEAD - NBUF).wait()
                x_copy(t + AHEAD).start()

        for t in range(max(0, total - NBUF), total):
            o_copy(t).wait()

    out = pl.pallas_call(
        body,
        in_specs=[
            pl.BlockSpec(memory_space=pl.ANY),
            pl.BlockSpec(memory_space=pl.ANY),
        ],
        out_specs=pl.BlockSpec(memory_space=pl.ANY),
        out_shape=jax.ShapeDtypeStruct((rows, d), x.dtype),
        scratch_shapes=[
            pltpu.VMEM((NBUF, CHUNK, d), jnp.float32),
            pltpu.VMEM((2, CHUNK, d), jnp.float32),
            pltpu.SemaphoreType.DMA((NBUF,)),
            pltpu.SemaphoreType.DMA((NBUF,)),
            pltpu.SemaphoreType.DMA((2,)),
        ],
        compiler_params=pltpu.CompilerParams(
            vmem_limit_bytes=64 * 1024 * 1024,
        ),
    )(x2, pe2)
    return out.reshape(batch, seq_len, d)
